# Initial kernel scaffold; baseline (speedup 1.0000x reference)
import functools

import jax
import jax.numpy as jnp
from jax.experimental import pallas as pl
from jax.experimental.pallas import tpu as pltpu
from jax.experimental.pallas import tpu_sc as plsc
from jax import lax

NUM_EMBEDDINGS = 1000000
D = 64
B_TOTAL = 16384 * 50

NC = 2
NS = 16
NW = NC * NS

B_PER_W = B_TOTAL // NW
CHUNK = 800
N_CHUNKS = B_PER_W // CHUNK


def _gather_body(ids_hbm, table_hbm, out_hbm, idx_v, rows_v, sem_g, sem_o):
    wid = lax.axis_index("s") * NC + lax.axis_index("c")
    base = wid * B_PER_W

    @pl.loop(0, N_CHUNKS)
    def _chunk(g):
        off = base + g * CHUNK
        pltpu.sync_copy(ids_hbm.at[pl.ds(off, CHUNK)], idx_v)
        pltpu.async_copy(table_hbm.at[idx_v], rows_v, sem_g).wait()
        pltpu.async_copy(rows_v, out_hbm.at[pl.ds(off, CHUNK)], sem_o).wait()


@jax.jit
def _embedding_gather(ids_flat, table):
    mesh = plsc.VectorSubcoreMesh(
        core_axis_name="c", subcore_axis_name="s", num_cores=NC, num_subcores=NS
    )
    return pl.kernel(
        _gather_body,
        out_type=jax.ShapeDtypeStruct((B_TOTAL, D), jnp.float32),
        mesh=mesh,
        scratch_types=[
            pltpu.VMEM((CHUNK,), jnp.int32),
            pltpu.VMEM((CHUNK, D), jnp.float32),
            pltpu.SemaphoreType.DMA,
            pltpu.SemaphoreType.DMA,
        ],
    )(ids_flat, table)


def kernel(input_ids, attention_mask, table):
    ids_flat = input_ids.reshape(-1).astype(jnp.int32)
    out = _embedding_gather(ids_flat, table)
    return out.reshape(input_ids.shape + (D,))


# speedup vs baseline: 1.8300x; 1.8300x over previous
import functools

import jax
import jax.numpy as jnp
from jax.experimental import pallas as pl
from jax.experimental.pallas import tpu as pltpu
from jax.experimental.pallas import tpu_sc as plsc
from jax import lax

NUM_EMBEDDINGS = 1000000
D = 64
B_TOTAL = 16384 * 50

NC = 2
NS = 16
NW = NC * NS

B_PER_W = B_TOTAL // NW
CHUNK = 800
N_CHUNKS = B_PER_W // CHUNK


def _gather_body(ids_hbm, table_hbm, out_hbm, idx_v, rows_v, sem_g, sem_o):
    wid = lax.axis_index("s") * NC + lax.axis_index("c")
    base = wid * B_PER_W

    @pl.loop(0, N_CHUNKS)
    def _chunk(g):
        off = base + g * CHUNK
        pltpu.sync_copy(ids_hbm.at[pl.ds(off, CHUNK)], idx_v)
        pltpu.async_copy(table_hbm.at[idx_v], rows_v, sem_g).wait()
        pltpu.async_copy(rows_v, out_hbm.at[pl.ds(off, CHUNK)], sem_o).wait()


@jax.jit
def _embedding_gather(ids_flat, table):
    mesh = plsc.VectorSubcoreMesh(
        core_axis_name="c", subcore_axis_name="s", num_cores=NC, num_subcores=NS
    )
    return pl.kernel(
        _gather_body,
        out_type=jax.ShapeDtypeStruct((B_TOTAL, D), jnp.float32),
        mesh=mesh,
        scratch_types=[
            pltpu.VMEM((CHUNK,), jnp.int32),
            pltpu.VMEM((CHUNK, D), jnp.float32),
            pltpu.SemaphoreType.DMA,
            pltpu.SemaphoreType.DMA,
        ],
        compiler_params=pltpu.CompilerParams(use_tc_tiling_on_sc=False),
    )(ids_flat, table)


def kernel(input_ids, attention_mask, table):
    ids_flat = input_ids.reshape(-1).astype(jnp.int32)
    out = _embedding_gather(ids_flat, table)
    return out.reshape(input_ids.shape + (D,))


# speedup vs baseline: 1.8742x; 1.0241x over previous
import jax
import jax.numpy as jnp
from jax.experimental import pallas as pl
from jax.experimental.pallas import tpu as pltpu
from jax.experimental.pallas import tpu_sc as plsc
from jax import lax

NUM_EMBEDDINGS = 1000000
D = 64
B_TOTAL = 16384 * 50

NC = 2
NS = 16
NW = NC * NS

B_PER_W = B_TOTAL // NW
CHUNK = 800
N_CHUNKS = B_PER_W // CHUNK


def _gather_body(ids_hbm, table_hbm, out_hbm,
                 idx0, idx1, idx2, idx3, rows0, rows1,
                 si0, si1, si2, si3, sg0, sg1, so0, so1):
    idx = [idx0, idx1, idx2, idx3]
    sem_i = [si0, si1, si2, si3]
    rows = [rows0, rows1]
    sem_g = [sg0, sg1]
    sem_o = [so0, so1]

    wid = lax.axis_index("s") * NC + lax.axis_index("c")
    base = wid * B_PER_W

    def ids_at(g):
        return ids_hbm.at[pl.ds(base + g * CHUNK, CHUNK)]

    def out_at(g):
        return out_hbm.at[pl.ds(base + g * CHUNK, CHUNK)]

    for b in range(4):
        pltpu.async_copy(ids_at(b), idx[b], sem_i[b])

    @pl.loop(0, N_CHUNKS, step=4)
    def _steps(g0):
        for b in range(4):
            g = g0 + b
            rb = b % 2
            pltpu.make_async_copy(ids_at(g), idx[b], sem_i[b]).wait()

            @pl.when(g >= 2)
            def _():
                pltpu.make_async_copy(rows[rb], out_at(g), sem_o[rb]).wait()

            pltpu.async_copy(table_hbm.at[idx[b]], rows[rb], sem_g[rb])

            @pl.when(g >= 1)
            def _():
                pb = (b + 1) % 2
                ib = (b + 3) % 4
                pltpu.make_async_copy(
                    table_hbm.at[idx[ib]], rows[pb], sem_g[pb]).wait()
                pltpu.async_copy(rows[pb], out_at(g - 1), sem_o[pb])

                @pl.when(g + 3 < N_CHUNKS)
                def _():
                    pltpu.async_copy(ids_at(g + 3), idx[ib], sem_i[ib])

    last = N_CHUNKS - 1
    pltpu.make_async_copy(table_hbm.at[idx[3]], rows[1], sem_g[1]).wait()
    pltpu.async_copy(rows[1], out_at(last), sem_o[1])
    pltpu.make_async_copy(rows[0], out_at(last - 1), sem_o[0]).wait()
    pltpu.make_async_copy(rows[1], out_at(last), sem_o[1]).wait()


@jax.jit
def _embedding_gather(ids_flat, table):
    mesh = plsc.VectorSubcoreMesh(
        core_axis_name="c", subcore_axis_name="s", num_cores=NC, num_subcores=NS
    )
    return pl.kernel(
        _gather_body,
        out_type=jax.ShapeDtypeStruct((B_TOTAL, D), jnp.float32),
        mesh=mesh,
        scratch_types=[
            pltpu.VMEM((CHUNK,), jnp.int32),
            pltpu.VMEM((CHUNK,), jnp.int32),
            pltpu.VMEM((CHUNK,), jnp.int32),
            pltpu.VMEM((CHUNK,), jnp.int32),
            pltpu.VMEM((CHUNK, D), jnp.float32),
            pltpu.VMEM((CHUNK, D), jnp.float32),
            pltpu.SemaphoreType.DMA,
            pltpu.SemaphoreType.DMA,
            pltpu.SemaphoreType.DMA,
            pltpu.SemaphoreType.DMA,
            pltpu.SemaphoreType.DMA,
            pltpu.SemaphoreType.DMA,
            pltpu.SemaphoreType.DMA,
            pltpu.SemaphoreType.DMA,
        ],
        compiler_params=pltpu.CompilerParams(use_tc_tiling_on_sc=False),
    )(ids_flat, table)


def kernel(input_ids, attention_mask, table):
    ids_flat = input_ids.reshape(-1).astype(jnp.int32)
    out = _embedding_gather(ids_flat, table)
    return out.reshape(input_ids.shape + (D,))
